# baseline (speedup 1.0000x reference)
import functools
import math

import jax
import jax.numpy as jnp
from jax import lax
from jax.experimental import pallas as pl
from jax.experimental.pallas import tpu as pltpu
from jax.experimental.pallas import tpu_sc as plsc

_SMOOTH = 0.1
_VOCAB = 32000
_EPS = _SMOOTH / (_VOCAB - 1)
_CONF = 1.0 - _SMOOTH
_CD = _CONF - _EPS
_C = (_VOCAB - 1) * _EPS * math.log(_EPS) + _CONF * math.log(_CONF)

_NC, _NS, _L = 2, 16, 16
_NW = _NC * _NS


def _tc_body(pred_ref, tgt_ref, s_ref, n_ref):
    i = pl.program_id(0)
    m = (tgt_ref[...] > 0).astype(jnp.float32)
    rs = jnp.sum(pred_ref[...], axis=1, keepdims=True)
    part = jnp.sum(rs * m)

    @pl.when(i == 0)
    def _():
        s_ref[...] = jnp.zeros_like(s_ref)
        n_ref[...] = jnp.sum(m).reshape(1, 1)

    s_ref[...] += part.reshape(1, 1)


def _sc_body(bpw, pred_hbm, tgt_hbm, out_hbm, tgt_v, idx_v, g_v, acc_v, sem):
    wid = lax.axis_index("s") * _NC + lax.axis_index("c")
    base = wid * bpw
    pltpu.sync_copy(tgt_hbm.at[pl.ds(base, bpw)], tgt_v)
    for j in range(bpw // _L):
        t = tgt_v[pl.ds(j * _L, _L)]
        rows = lax.iota(jnp.int32, _L) + (base + j * _L)
        idx_v[pl.ds(j * _L, _L)] = rows * _VOCAB + t
    pltpu.async_copy(pred_hbm.at[idx_v], g_v, sem).wait()
    acc = jnp.zeros((_L,), jnp.float32)
    for j in range(bpw // _L):
        t = tgt_v[pl.ds(j * _L, _L)]
        g = g_v[pl.ds(j * _L, _L)]
        acc = acc + jnp.where(t > 0, g, jnp.float32(0.0))
    acc_v[...] = acc
    pltpu.sync_copy(acc_v, out_hbm.at[wid])


def kernel(prediction, target):
    rows = prediction.shape[0] * prediction.shape[1]
    vocab = prediction.shape[-1]
    pred = prediction.reshape(rows, vocab)
    tgt = target.reshape(rows, 1).astype(jnp.int32)
    tgt_flat = target.reshape(rows).astype(jnp.int32)
    bpw = rows // _NW

    mesh = plsc.VectorSubcoreMesh(core_axis_name="c", subcore_axis_name="s")
    sc_gather = functools.partial(
        pl.kernel,
        mesh=mesh,
        out_type=jax.ShapeDtypeStruct((_NW, _L), jnp.float32),
        scratch_types=[
            pltpu.VMEM((bpw,), jnp.int32),
            pltpu.VMEM((bpw,), jnp.int32),
            pltpu.VMEM((bpw,), jnp.float32),
            pltpu.VMEM((_L,), jnp.float32),
            pltpu.SemaphoreType.DMA,
        ],
    )(functools.partial(_sc_body, bpw))
    g_parts = sc_gather(pred.reshape(rows * vocab), tgt_flat)

    col_block = 1280
    grid = (vocab // col_block,)
    s, n = pl.pallas_call(
        _tc_body,
        grid=grid,
        in_specs=[
            pl.BlockSpec((rows, col_block), lambda i: (0, i)),
            pl.BlockSpec((rows, 1), lambda i: (0, 0)),
        ],
        out_specs=[
            pl.BlockSpec((1, 1), lambda i: (0, 0)),
            pl.BlockSpec((1, 1), lambda i: (0, 0)),
        ],
        out_shape=[
            jax.ShapeDtypeStruct((1, 1), jnp.float32),
            jax.ShapeDtypeStruct((1, 1), jnp.float32),
        ],
    )(pred, tgt)

    s2 = jnp.sum(g_parts)
    nval = n[0, 0]
    total = _EPS * s[0, 0] + _CD * s2
    return jnp.float32(_C) - total / nval


# speedup vs baseline: 3.2022x; 3.2022x over previous
import functools
import math

import jax
import jax.numpy as jnp
from jax.experimental import pallas as pl
from jax.experimental.pallas import tpu as pltpu

_SMOOTH = 0.1
_VOCAB = 32000
_EPS = _SMOOTH / (_VOCAB - 1)
_CONF = 1.0 - _SMOOTH
_CD = _CONF - _EPS
_C = (_VOCAB - 1) * _EPS * math.log(_EPS) + _CONF * math.log(_CONF)


def _tc_body(col_block, pred_ref, tgt_ref, s_ref, n_ref):
    i = pl.program_id(0)
    t = tgt_ref[...]
    m = (t > 0).astype(jnp.float32)
    p = pred_ref[...]
    cols = jax.lax.broadcasted_iota(jnp.int32, p.shape, 1)
    sel = jnp.where(cols == t - i * col_block, p, 0.0)
    rs = jnp.sum(p, axis=1, keepdims=True)
    ss = jnp.sum(sel, axis=1, keepdims=True)
    part = jnp.sum((_EPS * rs + _CD * ss) * m)

    @pl.when(i == 0)
    def _():
        s_ref[...] = jnp.zeros_like(s_ref)
        n_ref[...] = jnp.sum(m).reshape(1, 1)

    s_ref[...] += part.reshape(1, 1)


def kernel(prediction, target):
    rows = prediction.shape[0] * prediction.shape[1]
    vocab = prediction.shape[-1]
    pred = prediction.reshape(rows, vocab)
    tgt = target.reshape(rows, 1).astype(jnp.int32)

    col_block = 1280
    grid = (vocab // col_block,)
    s, n = pl.pallas_call(
        functools.partial(_tc_body, col_block),
        grid=grid,
        in_specs=[
            pl.BlockSpec((rows, col_block), lambda i: (0, i)),
            pl.BlockSpec((rows, 1), lambda i: (0, 0)),
        ],
        out_specs=[
            pl.BlockSpec((1, 1), lambda i: (0, 0)),
            pl.BlockSpec((1, 1), lambda i: (0, 0)),
        ],
        out_shape=[
            jax.ShapeDtypeStruct((1, 1), jnp.float32),
            jax.ShapeDtypeStruct((1, 1), jnp.float32),
        ],
    )(pred, tgt)
    nval = n[0, 0]
    return jnp.float32(_C) - s[0, 0] / nval


# speedup vs baseline: 3.2308x; 1.0089x over previous
import functools
import math

import jax
import jax.numpy as jnp
from jax.experimental import pallas as pl
from jax.experimental.pallas import tpu as pltpu

_SMOOTH = 0.1
_VOCAB = 32000
_EPS = _SMOOTH / (_VOCAB - 1)
_CONF = 1.0 - _SMOOTH
_CD = _CONF - _EPS
_C = (_VOCAB - 1) * _EPS * math.log(_EPS) + _CONF * math.log(_CONF)


def _tc_body(col_block, pred_ref, tgt_ref, s_ref, n_ref):
    i = pl.program_id(0)
    t = tgt_ref[...]
    m = (t > 0).astype(jnp.float32)
    p = pred_ref[...]
    cols = jax.lax.broadcasted_iota(jnp.int32, p.shape, 1)
    sel = jnp.where(cols == t - i * col_block, p, 0.0)
    rs = jnp.sum(p, axis=1, keepdims=True)
    ss = jnp.sum(sel, axis=1, keepdims=True)
    part = jnp.sum((_EPS * rs + _CD * ss) * m)

    @pl.when(i == 0)
    def _():
        s_ref[...] = jnp.zeros_like(s_ref)
        n_ref[...] = jnp.sum(m).reshape(1, 1)

    s_ref[...] += part.reshape(1, 1)


def kernel(prediction, target):
    rows = prediction.shape[0] * prediction.shape[1]
    vocab = prediction.shape[-1]
    pred = prediction.reshape(rows, vocab)
    tgt = target.reshape(rows, 1).astype(jnp.int32)

    col_block = 3200
    grid = (vocab // col_block,)
    s, n = pl.pallas_call(
        functools.partial(_tc_body, col_block),
        grid=grid,
        in_specs=[
            pl.BlockSpec((rows, col_block), lambda i: (0, i)),
            pl.BlockSpec((rows, 1), lambda i: (0, 0)),
        ],
        out_specs=[
            pl.BlockSpec((1, 1), lambda i: (0, 0)),
            pl.BlockSpec((1, 1), lambda i: (0, 0)),
        ],
        out_shape=[
            jax.ShapeDtypeStruct((1, 1), jnp.float32),
            jax.ShapeDtypeStruct((1, 1), jnp.float32),
        ],
    )(pred, tgt)
    nval = n[0, 0]
    return jnp.float32(_C) - s[0, 0] / nval
